# baseline (speedup 1.0000x reference)
import jax
import jax.numpy as jnp
from jax import lax
from jax.experimental import pallas as pl
from jax.experimental.pallas import tpu as pltpu
from jax.experimental.pallas import tpu_sc as plsc

B, D, N, M = 4, 64, 4096, 4096
BM = 4096

NUM_WORKERS = 32
LANES = 16
CHUNK = (B * N) // NUM_WORKERS
VECS = CHUNK // LANES
ROWS_PER_BATCH = N // CHUNK


SLAB = 8


RC = 128


def _argmax_into(at_ref, b_ref, idx_ref, val_ref):
    def step(i, carry):
        run_max, run_slab = carry
        a_chunk = at_ref[0, :, pl.ds(pl.multiple_of(i * RC, RC), RC)]
        s = lax.dot_general(
            a_chunk, b_ref[0],
            dimension_numbers=(((0,), (0,)), ((), ())),
            preferred_element_type=jnp.float32,
        )
        for k in range(RC // SLAB):
            v = s[k * SLAB:(k + 1) * SLAB]
            slab_id = (i * (RC // SLAB) + k).astype(jnp.float32)
            better = v > run_max
            run_max = jnp.maximum(v, run_max)
            run_slab = jnp.where(better, slab_id, run_slab)
        return (run_max, run_slab)
    init = (jnp.full((SLAB, BM), -jnp.inf, jnp.float32),
            jnp.zeros((SLAB, BM), jnp.float32))
    run_max, run_slab = lax.fori_loop(0, N // RC, step, init)

    sub = lax.broadcasted_iota(jnp.int32, (SLAB, BM), 0).astype(jnp.float32)
    grow = run_slab * jnp.float32(SLAB) + sub
    mx = jnp.max(run_max, axis=0)
    cand = jnp.where(run_max == mx[None, :], grow, jnp.float32(N))
    idx_ref[0, 0, :] = jnp.min(cand, axis=0).astype(jnp.int32)
    val_ref[0, 0, :] = (mx + 1.0) * 0.5


def _both_directions_body(d0f_ref, d1f_ref, d0s_ref, d1s_ref,
                          m1_ref, s1_ref, m0_ref, s0_ref):
    d = pl.program_id(2)

    @pl.when(d == 0)
    def _():
        _argmax_into(d0f_ref, d1s_ref, m1_ref, s1_ref)

    @pl.when(d == 1)
    def _():
        _argmax_into(d1f_ref, d0s_ref, m0_ref, s0_ref)


def _matmul_argmax_both(d0, d1):
    grid = (B, M // BM, 2)
    outs = pl.pallas_call(
        _both_directions_body,
        grid=grid,
        in_specs=[
            pl.BlockSpec((1, D, N), lambda bi, j, d: (bi, 0, 0)),
            pl.BlockSpec((1, D, N), lambda bi, j, d: (bi, 0, 0)),
            pl.BlockSpec((1, D, BM), lambda bi, j, d: (bi, 0, j)),
            pl.BlockSpec((1, D, BM), lambda bi, j, d: (bi, 0, j)),
        ],
        out_specs=[
            pl.BlockSpec((1, 1, BM), lambda bi, j, d: (bi, 0, j)),
            pl.BlockSpec((1, 1, BM), lambda bi, j, d: (bi, 0, j)),
            pl.BlockSpec((1, 1, BM), lambda bi, j, d: (bi, 0, j)),
            pl.BlockSpec((1, 1, BM), lambda bi, j, d: (bi, 0, j)),
        ],
        out_shape=[
            jax.ShapeDtypeStruct((B, 1, M), jnp.int32),
            jax.ShapeDtypeStruct((B, 1, M), jnp.float32),
            jax.ShapeDtypeStruct((B, 1, M), jnp.int32),
            jax.ShapeDtypeStruct((B, 1, M), jnp.float32),
        ],
        compiler_params=pltpu.CompilerParams(
            dimension_semantics=("parallel", "parallel", "arbitrary"),
        ),
    )(d0, d1, d0, d1)
    return outs


def _mutual_check_body(m0_hbm, m1_hbm, out_hbm, m1_v, m0_v, out_v):
    wid = lax.axis_index("s") * 2 + lax.axis_index("c")
    batch = wid // ROWS_PER_BATCH
    off = (wid % ROWS_PER_BATCH) * CHUNK
    pltpu.sync_copy(m1_hbm.at[batch, 0], m1_v)
    pltpu.sync_copy(m0_hbm.at[batch, 0, pl.ds(off, CHUNK)], m0_v)
    for i in range(VECS):
        idx = m0_v[pl.ds(i * LANES, LANES)]
        loop = plsc.load_gather(m1_v, [idx])
        inds = off + i * LANES + lax.iota(jnp.int32, LANES)
        out_v[pl.ds(i * LANES, LANES)] = jnp.where(loop == inds, idx, -1)
    pltpu.sync_copy(out_v, out_hbm.at[batch, pl.ds(off, CHUNK)])


def _mutual_check(m0, m1):
    run = pl.kernel(
        _mutual_check_body,
        mesh=plsc.VectorSubcoreMesh(core_axis_name="c", subcore_axis_name="s"),
        out_type=jax.ShapeDtypeStruct((B, N), jnp.int32),
        scratch_types=[
            pltpu.VMEM((M,), jnp.int32),
            pltpu.VMEM((CHUNK,), jnp.int32),
            pltpu.VMEM((CHUNK,), jnp.int32),
        ],
        compiler_params=pltpu.CompilerParams(needs_layout_passes=False),
    )
    return run(m0, m1)


@jax.jit
def kernel(descriptors0, descriptors1):
    matches1_3d, scores1_3d, matches0_3d, scores0_3d = _matmul_argmax_both(
        descriptors0, descriptors1)
    matches0 = _mutual_check(matches0_3d, matches1_3d)
    return (matches0, matches1_3d.reshape(B, M),
            scores0_3d.reshape(B, N), scores1_3d.reshape(B, M))


# speedup vs baseline: 1.4914x; 1.4914x over previous
import jax
import jax.numpy as jnp
from jax import lax
from jax.experimental import pallas as pl
from jax.experimental.pallas import tpu as pltpu
from jax.experimental.pallas import tpu_sc as plsc

B, D, N, M = 4, 64, 4096, 4096
BM = 4096

NUM_WORKERS = 32
LANES = 16
CHUNK = (B * N) // NUM_WORKERS
VECS = CHUNK // LANES
ROWS_PER_BATCH = N // CHUNK


SLAB = 8


RC = 512


def _argmax_into(at_ref, b_ref, idx_ref, val_ref):
    def step(i, carry):
        run_max, run_slab = carry
        a_chunk = at_ref[0, :, pl.ds(pl.multiple_of(i * RC, RC), RC)]
        s = lax.dot_general(
            a_chunk, b_ref[0],
            dimension_numbers=(((0,), (0,)), ((), ())),
            preferred_element_type=jnp.float32,
        )
        for k in range(RC // SLAB):
            v = s[k * SLAB:(k + 1) * SLAB]
            slab_id = (i * (RC // SLAB) + k).astype(jnp.float32)
            better = v > run_max
            run_max = jnp.maximum(v, run_max)
            run_slab = jnp.where(better, slab_id, run_slab)
        return (run_max, run_slab)
    init = (jnp.full((SLAB, BM), -jnp.inf, jnp.float32),
            jnp.zeros((SLAB, BM), jnp.float32))
    run_max, run_slab = lax.fori_loop(0, N // RC, step, init)

    sub = lax.broadcasted_iota(jnp.int32, (SLAB, BM), 0).astype(jnp.float32)
    grow = run_slab * jnp.float32(SLAB) + sub
    mx = jnp.max(run_max, axis=0)
    cand = jnp.where(run_max == mx[None, :], grow, jnp.float32(N))
    idx_ref[0, 0, :] = jnp.min(cand, axis=0).astype(jnp.int32)
    val_ref[0, 0, :] = (mx + 1.0) * 0.5


def _both_directions_body(d0f_ref, d1f_ref, d0s_ref, d1s_ref,
                          m1_ref, s1_ref, m0_ref, s0_ref):
    d = pl.program_id(2)

    @pl.when(d == 0)
    def _():
        _argmax_into(d0f_ref, d1s_ref, m1_ref, s1_ref)

    @pl.when(d == 1)
    def _():
        _argmax_into(d1f_ref, d0s_ref, m0_ref, s0_ref)


def _matmul_argmax_both(d0, d1):
    grid = (B, M // BM, 2)
    outs = pl.pallas_call(
        _both_directions_body,
        grid=grid,
        in_specs=[
            pl.BlockSpec((1, D, N), lambda bi, j, d: (bi, 0, 0)),
            pl.BlockSpec((1, D, N), lambda bi, j, d: (bi, 0, 0)),
            pl.BlockSpec((1, D, BM), lambda bi, j, d: (bi, 0, j)),
            pl.BlockSpec((1, D, BM), lambda bi, j, d: (bi, 0, j)),
        ],
        out_specs=[
            pl.BlockSpec((1, 1, BM), lambda bi, j, d: (bi, 0, j)),
            pl.BlockSpec((1, 1, BM), lambda bi, j, d: (bi, 0, j)),
            pl.BlockSpec((1, 1, BM), lambda bi, j, d: (bi, 0, j)),
            pl.BlockSpec((1, 1, BM), lambda bi, j, d: (bi, 0, j)),
        ],
        out_shape=[
            jax.ShapeDtypeStruct((B, 1, M), jnp.int32),
            jax.ShapeDtypeStruct((B, 1, M), jnp.float32),
            jax.ShapeDtypeStruct((B, 1, M), jnp.int32),
            jax.ShapeDtypeStruct((B, 1, M), jnp.float32),
        ],
        compiler_params=pltpu.CompilerParams(
            dimension_semantics=("parallel", "parallel", "arbitrary"),
        ),
    )(d0, d1, d0, d1)
    return outs


def _mutual_check_body(m0_hbm, m1_hbm, out_hbm, m1_v, m0_v, out_v):
    wid = lax.axis_index("s") * 2 + lax.axis_index("c")
    batch = wid // ROWS_PER_BATCH
    off = (wid % ROWS_PER_BATCH) * CHUNK
    pltpu.sync_copy(m1_hbm.at[batch, 0], m1_v)
    pltpu.sync_copy(m0_hbm.at[batch, 0, pl.ds(off, CHUNK)], m0_v)
    for i in range(VECS):
        idx = m0_v[pl.ds(i * LANES, LANES)]
        loop = plsc.load_gather(m1_v, [idx])
        inds = off + i * LANES + lax.iota(jnp.int32, LANES)
        out_v[pl.ds(i * LANES, LANES)] = jnp.where(loop == inds, idx, -1)
    pltpu.sync_copy(out_v, out_hbm.at[batch, pl.ds(off, CHUNK)])


def _mutual_check(m0, m1):
    run = pl.kernel(
        _mutual_check_body,
        mesh=plsc.VectorSubcoreMesh(core_axis_name="c", subcore_axis_name="s"),
        out_type=jax.ShapeDtypeStruct((B, N), jnp.int32),
        scratch_types=[
            pltpu.VMEM((M,), jnp.int32),
            pltpu.VMEM((CHUNK,), jnp.int32),
            pltpu.VMEM((CHUNK,), jnp.int32),
        ],
        compiler_params=pltpu.CompilerParams(needs_layout_passes=False),
    )
    return run(m0, m1)


@jax.jit
def kernel(descriptors0, descriptors1):
    matches1_3d, scores1_3d, matches0_3d, scores0_3d = _matmul_argmax_both(
        descriptors0, descriptors1)
    matches0 = _mutual_check(matches0_3d, matches1_3d)
    return (matches0, matches1_3d.reshape(B, M),
            scores0_3d.reshape(B, N), scores1_3d.reshape(B, M))


# speedup vs baseline: 1.6139x; 1.0821x over previous
import jax
import jax.numpy as jnp
from jax import lax
from jax.experimental import pallas as pl
from jax.experimental.pallas import tpu as pltpu
from jax.experimental.pallas import tpu_sc as plsc

B, D, N, M = 4, 64, 4096, 4096
BM = 4096

NUM_WORKERS = 32
LANES = 16
CHUNK = (B * N) // NUM_WORKERS
VECS = CHUNK // LANES
ROWS_PER_BATCH = N // CHUNK


SLAB = 8


RC = 1024


def _argmax_into(at_ref, b_ref, idx_ref, val_ref):
    def step(i, carry):
        run_max, run_slab = carry
        a_chunk = at_ref[0, :, pl.ds(pl.multiple_of(i * RC, RC), RC)]
        s = lax.dot_general(
            a_chunk, b_ref[0],
            dimension_numbers=(((0,), (0,)), ((), ())),
            preferred_element_type=jnp.float32,
        )
        for k in range(RC // SLAB):
            v = s[k * SLAB:(k + 1) * SLAB]
            slab_id = (i * (RC // SLAB) + k).astype(jnp.float32)
            better = v > run_max
            run_max = jnp.maximum(v, run_max)
            run_slab = jnp.where(better, slab_id, run_slab)
        return (run_max, run_slab)
    init = (jnp.full((SLAB, BM), -jnp.inf, jnp.float32),
            jnp.zeros((SLAB, BM), jnp.float32))
    run_max, run_slab = lax.fori_loop(0, N // RC, step, init)

    sub = lax.broadcasted_iota(jnp.int32, (SLAB, BM), 0).astype(jnp.float32)
    grow = run_slab * jnp.float32(SLAB) + sub
    mx = jnp.max(run_max, axis=0)
    cand = jnp.where(run_max == mx[None, :], grow, jnp.float32(N))
    idx_ref[0, 0, :] = jnp.min(cand, axis=0).astype(jnp.int32)
    val_ref[0, 0, :] = (mx + 1.0) * 0.5


def _both_directions_body(d0f_ref, d1f_ref, d0s_ref, d1s_ref,
                          m1_ref, s1_ref, m0_ref, s0_ref):
    d = pl.program_id(2)

    @pl.when(d == 0)
    def _():
        _argmax_into(d0f_ref, d1s_ref, m1_ref, s1_ref)

    @pl.when(d == 1)
    def _():
        _argmax_into(d1f_ref, d0s_ref, m0_ref, s0_ref)


def _matmul_argmax_both(d0, d1):
    grid = (B, M // BM, 2)
    outs = pl.pallas_call(
        _both_directions_body,
        grid=grid,
        in_specs=[
            pl.BlockSpec((1, D, N), lambda bi, j, d: (bi, 0, 0)),
            pl.BlockSpec((1, D, N), lambda bi, j, d: (bi, 0, 0)),
            pl.BlockSpec((1, D, BM), lambda bi, j, d: (bi, 0, j)),
            pl.BlockSpec((1, D, BM), lambda bi, j, d: (bi, 0, j)),
        ],
        out_specs=[
            pl.BlockSpec((1, 1, BM), lambda bi, j, d: (bi, 0, j)),
            pl.BlockSpec((1, 1, BM), lambda bi, j, d: (bi, 0, j)),
            pl.BlockSpec((1, 1, BM), lambda bi, j, d: (bi, 0, j)),
            pl.BlockSpec((1, 1, BM), lambda bi, j, d: (bi, 0, j)),
        ],
        out_shape=[
            jax.ShapeDtypeStruct((B, 1, M), jnp.int32),
            jax.ShapeDtypeStruct((B, 1, M), jnp.float32),
            jax.ShapeDtypeStruct((B, 1, M), jnp.int32),
            jax.ShapeDtypeStruct((B, 1, M), jnp.float32),
        ],
        compiler_params=pltpu.CompilerParams(
            dimension_semantics=("parallel", "parallel", "arbitrary"),
        ),
    )(d0, d1, d0, d1)
    return outs


def _mutual_check_body(m0_hbm, m1_hbm, out_hbm, m1_v, m0_v, out_v):
    wid = lax.axis_index("s") * 2 + lax.axis_index("c")
    batch = wid // ROWS_PER_BATCH
    off = (wid % ROWS_PER_BATCH) * CHUNK
    pltpu.sync_copy(m1_hbm.at[batch, 0], m1_v)
    pltpu.sync_copy(m0_hbm.at[batch, 0, pl.ds(off, CHUNK)], m0_v)
    for i in range(VECS):
        idx = m0_v[pl.ds(i * LANES, LANES)]
        loop = plsc.load_gather(m1_v, [idx])
        inds = off + i * LANES + lax.iota(jnp.int32, LANES)
        out_v[pl.ds(i * LANES, LANES)] = jnp.where(loop == inds, idx, -1)
    pltpu.sync_copy(out_v, out_hbm.at[batch, pl.ds(off, CHUNK)])


def _mutual_check(m0, m1):
    run = pl.kernel(
        _mutual_check_body,
        mesh=plsc.VectorSubcoreMesh(core_axis_name="c", subcore_axis_name="s"),
        out_type=jax.ShapeDtypeStruct((B, N), jnp.int32),
        scratch_types=[
            pltpu.VMEM((M,), jnp.int32),
            pltpu.VMEM((CHUNK,), jnp.int32),
            pltpu.VMEM((CHUNK,), jnp.int32),
        ],
        compiler_params=pltpu.CompilerParams(needs_layout_passes=False),
    )
    return run(m0, m1)


@jax.jit
def kernel(descriptors0, descriptors1):
    matches1_3d, scores1_3d, matches0_3d, scores0_3d = _matmul_argmax_both(
        descriptors0, descriptors1)
    matches0 = _mutual_check(matches0_3d, matches1_3d)
    return (matches0, matches1_3d.reshape(B, M),
            scores0_3d.reshape(B, N), scores1_3d.reshape(B, M))


# speedup vs baseline: 1.6820x; 1.0422x over previous
import jax
import jax.numpy as jnp
from jax import lax
from jax.experimental import pallas as pl
from jax.experimental.pallas import tpu as pltpu
from jax.experimental.pallas import tpu_sc as plsc

B, D, N, M = 4, 64, 4096, 4096
BM = 4096

NUM_WORKERS = 32
LANES = 16
CHUNK = (B * N) // NUM_WORKERS
VECS = CHUNK // LANES
ROWS_PER_BATCH = N // CHUNK


SLAB = 8


RC = 2048


def _argmax_into(at_ref, b_ref, idx_ref, val_ref):
    def step(i, carry):
        run_max, run_slab = carry
        a_chunk = at_ref[0, :, pl.ds(pl.multiple_of(i * RC, RC), RC)]
        s = lax.dot_general(
            a_chunk, b_ref[0],
            dimension_numbers=(((0,), (0,)), ((), ())),
            preferred_element_type=jnp.float32,
        )
        for k in range(RC // SLAB):
            v = s[k * SLAB:(k + 1) * SLAB]
            slab_id = (i * (RC // SLAB) + k).astype(jnp.float32)
            better = v > run_max
            run_max = jnp.maximum(v, run_max)
            run_slab = jnp.where(better, slab_id, run_slab)
        return (run_max, run_slab)
    init = (jnp.full((SLAB, BM), -jnp.inf, jnp.float32),
            jnp.zeros((SLAB, BM), jnp.float32))
    run_max, run_slab = lax.fori_loop(0, N // RC, step, init)

    sub = lax.broadcasted_iota(jnp.int32, (SLAB, BM), 0).astype(jnp.float32)
    grow = run_slab * jnp.float32(SLAB) + sub
    mx = jnp.max(run_max, axis=0)
    cand = jnp.where(run_max == mx[None, :], grow, jnp.float32(N))
    idx_ref[0, 0, :] = jnp.min(cand, axis=0).astype(jnp.int32)
    val_ref[0, 0, :] = (mx + 1.0) * 0.5


def _both_directions_body(d0f_ref, d1f_ref, d0s_ref, d1s_ref,
                          m1_ref, s1_ref, m0_ref, s0_ref):
    d = pl.program_id(2)

    @pl.when(d == 0)
    def _():
        _argmax_into(d0f_ref, d1s_ref, m1_ref, s1_ref)

    @pl.when(d == 1)
    def _():
        _argmax_into(d1f_ref, d0s_ref, m0_ref, s0_ref)


def _matmul_argmax_both(d0, d1):
    grid = (B, M // BM, 2)
    outs = pl.pallas_call(
        _both_directions_body,
        grid=grid,
        in_specs=[
            pl.BlockSpec((1, D, N), lambda bi, j, d: (bi, 0, 0)),
            pl.BlockSpec((1, D, N), lambda bi, j, d: (bi, 0, 0)),
            pl.BlockSpec((1, D, BM), lambda bi, j, d: (bi, 0, j)),
            pl.BlockSpec((1, D, BM), lambda bi, j, d: (bi, 0, j)),
        ],
        out_specs=[
            pl.BlockSpec((1, 1, BM), lambda bi, j, d: (bi, 0, j)),
            pl.BlockSpec((1, 1, BM), lambda bi, j, d: (bi, 0, j)),
            pl.BlockSpec((1, 1, BM), lambda bi, j, d: (bi, 0, j)),
            pl.BlockSpec((1, 1, BM), lambda bi, j, d: (bi, 0, j)),
        ],
        out_shape=[
            jax.ShapeDtypeStruct((B, 1, M), jnp.int32),
            jax.ShapeDtypeStruct((B, 1, M), jnp.float32),
            jax.ShapeDtypeStruct((B, 1, M), jnp.int32),
            jax.ShapeDtypeStruct((B, 1, M), jnp.float32),
        ],
        compiler_params=pltpu.CompilerParams(
            dimension_semantics=("parallel", "parallel", "arbitrary"),
        ),
    )(d0, d1, d0, d1)
    return outs


def _mutual_check_body(m0_hbm, m1_hbm, out_hbm, m1_v, m0_v, out_v):
    wid = lax.axis_index("s") * 2 + lax.axis_index("c")
    batch = wid // ROWS_PER_BATCH
    off = (wid % ROWS_PER_BATCH) * CHUNK
    pltpu.sync_copy(m1_hbm.at[batch, 0], m1_v)
    pltpu.sync_copy(m0_hbm.at[batch, 0, pl.ds(off, CHUNK)], m0_v)
    for i in range(VECS):
        idx = m0_v[pl.ds(i * LANES, LANES)]
        loop = plsc.load_gather(m1_v, [idx])
        inds = off + i * LANES + lax.iota(jnp.int32, LANES)
        out_v[pl.ds(i * LANES, LANES)] = jnp.where(loop == inds, idx, -1)
    pltpu.sync_copy(out_v, out_hbm.at[batch, pl.ds(off, CHUNK)])


def _mutual_check(m0, m1):
    run = pl.kernel(
        _mutual_check_body,
        mesh=plsc.VectorSubcoreMesh(core_axis_name="c", subcore_axis_name="s"),
        out_type=jax.ShapeDtypeStruct((B, N), jnp.int32),
        scratch_types=[
            pltpu.VMEM((M,), jnp.int32),
            pltpu.VMEM((CHUNK,), jnp.int32),
            pltpu.VMEM((CHUNK,), jnp.int32),
        ],
        compiler_params=pltpu.CompilerParams(needs_layout_passes=False),
    )
    return run(m0, m1)


@jax.jit
def kernel(descriptors0, descriptors1):
    matches1_3d, scores1_3d, matches0_3d, scores0_3d = _matmul_argmax_both(
        descriptors0, descriptors1)
    matches0 = _mutual_check(matches0_3d, matches1_3d)
    return (matches0, matches1_3d.reshape(B, M),
            scores0_3d.reshape(B, N), scores1_3d.reshape(B, M))
